# baseline (speedup 1.0000x reference)
import functools

import jax
import jax.numpy as jnp
from jax import lax
from jax.experimental import pallas as pl
from jax.experimental.pallas import tpu as pltpu
from jax.experimental.pallas import tpu_sc as plsc

NC = 2
NS = 16
NW = NC * NS
LANES = 16


@functools.partial(jax.jit, static_argnums=(3, 4, 5, 6))
def _emb_lookup(x2d, table, pos, B, S, V, D):
    HS = S // 2
    seqs_per_w = B // NW

    mesh = plsc.VectorSubcoreMesh(core_axis_name="c", subcore_axis_name="s")

    @functools.partial(
        pl.kernel,
        out_type=jax.ShapeDtypeStruct((B * S, D), jnp.float32),
        mesh=mesh,
        scratch_types=[
            pltpu.VMEM((2 * seqs_per_w, HS), jnp.int32),
            pltpu.VMEM((S, D), jnp.float32),
            pltpu.VMEM((S, D), jnp.float32),
            pltpu.SemaphoreType.DMA,
        ],
        compiler_params=pltpu.CompilerParams(use_tc_tiling_on_sc=False),
    )
    def k(x_hbm, table_hbm, pos_hbm, out_hbm, idx_v, pos_v, rows_v, sem):
        wid = lax.axis_index("s") * NC + lax.axis_index("c")
        pltpu.sync_copy(x_hbm.at[pl.ds(wid * 2 * seqs_per_w, 2 * seqs_per_w)], idx_v)
        pltpu.sync_copy(pos_hbm, pos_v)

        def seq_body(i, _):
            seq = wid * seqs_per_w + i
            c1 = pltpu.async_copy(
                table_hbm.at[idx_v.at[2 * i]], rows_v.at[pl.ds(0, HS)], sem)
            c2 = pltpu.async_copy(
                table_hbm.at[idx_v.at[2 * i + 1]], rows_v.at[pl.ds(HS, HS)], sem)
            c1.wait()
            c2.wait()

            def row_body(r, _):
                for d in range(0, D, LANES):
                    rows_v[r, pl.ds(d, LANES)] = (
                        rows_v[r, pl.ds(d, LANES)] + pos_v[r, pl.ds(d, LANES)])
                return 0

            lax.fori_loop(0, S, row_body, 0, unroll=2)
            pltpu.sync_copy(rows_v, out_hbm.at[pl.ds(seq * S, S)])
            return 0

        lax.fori_loop(0, seqs_per_w, seq_body, 0)

    return k(x2d, table, pos)


def kernel(x, table, pos_enc):
    B, S = x.shape
    V, D = table.shape
    x2d = x.astype(jnp.int32).reshape(B * S // (S // 2), S // 2)
    pos = pos_enc[0, :S, :]
    out = _emb_lookup(x2d, table, pos, B, S, V, D)
    return out.reshape(B, S, D)


# speedup vs baseline: 1.3995x; 1.3995x over previous
import functools

import jax
import jax.numpy as jnp
from jax import lax
from jax.experimental import pallas as pl
from jax.experimental.pallas import tpu as pltpu
from jax.experimental.pallas import tpu_sc as plsc

NC = 2
NS = 16
NW = NC * NS
LANES = 16
NBUF = 4


@functools.partial(jax.jit, static_argnums=(3, 4, 5, 6))
def _emb_lookup(x2d, table, pos, B, S, V, D):
    HS = S // 2
    seqs_per_w = B // NW

    mesh = plsc.VectorSubcoreMesh(core_axis_name="c", subcore_axis_name="s")

    @functools.partial(
        pl.kernel,
        out_type=jax.ShapeDtypeStruct((B * S, D), jnp.float32),
        mesh=mesh,
        scratch_types=[
            pltpu.VMEM((2 * seqs_per_w, HS), jnp.int32),
            pltpu.VMEM((S, D), jnp.float32),
            pltpu.VMEM((NBUF, S, D), jnp.float32),
            pltpu.SemaphoreType.DMA,
            pltpu.SemaphoreType.DMA,
        ],
        compiler_params=pltpu.CompilerParams(use_tc_tiling_on_sc=False),
    )
    def k(x_hbm, table_hbm, pos_hbm, out_hbm, idx_v, pos_v, rows_v, gsem, osem):
        wid = lax.axis_index("s") * NC + lax.axis_index("c")
        base_seq = wid * seqs_per_w
        pltpu.sync_copy(x_hbm.at[pl.ds(wid * 2 * seqs_per_w, 2 * seqs_per_w)], idx_v)
        pltpu.sync_copy(pos_hbm, pos_v)

        def gather_parts(i):
            slot = lax.rem(i, NBUF)
            return (
                (table_hbm.at[idx_v.at[2 * i]], rows_v.at[slot, pl.ds(0, HS)]),
                (table_hbm.at[idx_v.at[2 * i + 1]], rows_v.at[slot, pl.ds(HS, HS)]),
            )

        def store_parts(i):
            slot = lax.rem(i, NBUF)
            return rows_v.at[slot], out_hbm.at[pl.ds((base_seq + i) * S, S)]

        def start_gather(i):
            for src, dst in gather_parts(i):
                pltpu.async_copy(src, dst, gsem)

        for p in range(NBUF - 1):
            start_gather(p)

        @pl.loop(0, seqs_per_w)
        def _(i):
            slot = lax.rem(i, NBUF)
            for src, dst in gather_parts(i):
                pltpu.make_async_copy(src, dst, gsem).wait()

            @pl.when(i >= 1)
            def _():
                src, dst = store_parts(i - 1)
                pltpu.make_async_copy(src, dst, osem).wait()

            @pl.when(i + NBUF - 1 < seqs_per_w)
            def _():
                start_gather(i + NBUF - 1)

            @plsc.parallel_loop(0, S, unroll=4)
            def _(r):
                for d in range(0, D, LANES):
                    rows_v[slot, r, pl.ds(d, LANES)] = (
                        rows_v[slot, r, pl.ds(d, LANES)] + pos_v[r, pl.ds(d, LANES)])

            src, dst = store_parts(i)
            pltpu.async_copy(src, dst, osem)

        src, dst = store_parts(seqs_per_w - 1)
        pltpu.make_async_copy(src, dst, osem).wait()

    return k(x2d, table, pos)


def kernel(x, table, pos_enc):
    B, S = x.shape
    V, D = table.shape
    x2d = x.astype(jnp.int32).reshape(B * S // (S // 2), S // 2)
    pos = pos_enc[0, :S, :]
    out = _emb_lookup(x2d, table, pos, B, S, V, D)
    return out.reshape(B, S, D)


# speedup vs baseline: 1.4015x; 1.0014x over previous
import functools

import jax
import jax.numpy as jnp
from jax import lax
from jax.experimental import pallas as pl
from jax.experimental.pallas import tpu as pltpu
from jax.experimental.pallas import tpu_sc as plsc

NC = 2
NS = 16
NW = NC * NS
LANES = 16
NBUF = 4


@functools.partial(jax.jit, static_argnums=(3, 4, 5, 6))
def _emb_lookup(x_flat, table_flat, pos, B, S, V, D):
    HS = S // 2
    seqs_per_w = B // NW

    mesh = plsc.VectorSubcoreMesh(core_axis_name="c", subcore_axis_name="s")

    @functools.partial(
        pl.kernel,
        out_type=jax.ShapeDtypeStruct((B, S, D), jnp.float32),
        mesh=mesh,
        scratch_types=[
            pltpu.VMEM((2 * seqs_per_w, HS), jnp.int32),
            pltpu.VMEM((S, D), jnp.float32),
            pltpu.VMEM((NBUF, S, D), jnp.float32),
            pltpu.SemaphoreType.DMA,
            pltpu.SemaphoreType.DMA,
        ],
        compiler_params=pltpu.CompilerParams(use_tc_tiling_on_sc=False),
    )
    def k(x_hbm, table_hbm, pos_hbm, out_hbm, idx_v, pos_v, rows_v, gsem, osem):
        wid = lax.axis_index("s") * NC + lax.axis_index("c")
        base_seq = wid * seqs_per_w
        x2d = x_hbm
        table2d = table_hbm
        out2d = out_hbm
        pltpu.sync_copy(x2d.at[pl.ds(wid * 2 * seqs_per_w, 2 * seqs_per_w)], idx_v)
        pltpu.sync_copy(pos_hbm, pos_v)

        def gather_parts(i):
            slot = lax.rem(i, NBUF)
            return (
                (table2d.at[idx_v.at[2 * i]], rows_v.at[slot, pl.ds(0, HS)]),
                (table2d.at[idx_v.at[2 * i + 1]], rows_v.at[slot, pl.ds(HS, HS)]),
            )

        def store_parts(i):
            slot = lax.rem(i, NBUF)
            return rows_v.at[slot], out2d.at[base_seq + i]

        def start_gather(i):
            for src, dst in gather_parts(i):
                pltpu.async_copy(src, dst, gsem)

        for p in range(NBUF - 1):
            start_gather(p)

        @pl.loop(0, seqs_per_w)
        def _(i):
            slot = lax.rem(i, NBUF)
            for src, dst in gather_parts(i):
                pltpu.make_async_copy(src, dst, gsem).wait()

            @pl.when(i >= 1)
            def _():
                src, dst = store_parts(i - 1)
                pltpu.make_async_copy(src, dst, osem).wait()

            @pl.when(i + NBUF - 1 < seqs_per_w)
            def _():
                start_gather(i + NBUF - 1)

            @plsc.parallel_loop(0, S, unroll=4)
            def _(r):
                for d in range(0, D, LANES):
                    rows_v[slot, r, pl.ds(d, LANES)] = (
                        rows_v[slot, r, pl.ds(d, LANES)] + pos_v[r, pl.ds(d, LANES)])

            src, dst = store_parts(i)
            pltpu.async_copy(src, dst, osem)

        src, dst = store_parts(seqs_per_w - 1)
        pltpu.make_async_copy(src, dst, osem).wait()

    return k(x_flat, table_flat, pos)


def kernel(x, table, pos_enc):
    B, S = x.shape
    V, D = table.shape
    x_flat = x.astype(jnp.int32).reshape(B * S // (S // 2), S // 2)
    table_flat = table
    pos = pos_enc[0, :S, :]
    out = _emb_lookup(x_flat, table_flat, pos, B, S, V, D)
    return out.reshape(B, S, D)
